# baseline (speedup 1.0000x reference)
import functools

import jax
import jax.numpy as jnp
from jax import lax
from jax.experimental import pallas as pl
from jax.experimental.pallas import tpu as pltpu
from jax.experimental.pallas import tpu_sc as plsc

NB = 4096
NT = 200
D = 64
DP = 128
NW = 32
BT = 128
NBT = NB // BT

_mesh = plsc.VectorSubcoreMesh(core_axis_name="c", subcore_axis_name="s")


@functools.partial(
    pl.kernel,
    mesh=_mesh,
    compiler_params=pltpu.CompilerParams(needs_layout_passes=False),
    out_type=jax.ShapeDtypeStruct((NT, D // 8, NBT, 8, BT), jnp.float32),
    scratch_types=[
        pltpu.VMEM((NT, BT), jnp.int32),
        pltpu.VMEM((4, BT, DP), jnp.float32),
        pltpu.VMEM((2, D // 8, 8, BT), jnp.float32),
        pltpu.SemaphoreType.DMA,
        pltpu.SemaphoreType.DMA,
    ],
)
def _emb_lookup(xt_hbm, table_hbm, out_hbm, idx_v, rows_v, tr_v,
                in_sem, out_sem):
    bt = lax.axis_index("s") * 2 + lax.axis_index("c")
    b0 = bt * BT
    pltpu.sync_copy(xt_hbm.at[:, pl.ds(b0, BT)], idx_v)

    lanes = lax.iota(jnp.int32, 16)

    def fire_gather(t, db):
        pltpu.async_copy(table_hbm.at[idx_v.at[t]], rows_v.at[db], in_sem)

    def wait_rows(db):
        pltpu.make_async_copy(
            table_hbm.at[pl.ds(0, BT)], rows_v.at[db], in_sem
        ).wait()

    def transpose(db4, db2):
        @plsc.parallel_loop(0, BT // 16, 1, unroll=2)
        def _(bg):
            b_vec = bg * 16 + lanes
            for dt in range(D // 8):
                for d_in in range(8):
                    d_vec = jnp.full((16,), dt * 8 + d_in, jnp.int32)
                    v = plsc.load_gather(rows_v.at[db4], [b_vec, d_vec])
                    tr_v[db2, dt, d_in, pl.ds(bg * 16, 16)] = v

    def wait_out(db):
        pltpu.make_async_copy(
            out_hbm.at[0, :, 0], tr_v.at[db], out_sem
        ).wait()

    def unit(t, db4, db2):
        wait_rows(db4)

        @pl.when(t + 3 < NT)
        def _():
            fire_gather(t + 3, (db4 + 3) % 4)

        @pl.when(t >= 2)
        def _():
            wait_out(db2)

        transpose(db4, db2)
        pltpu.async_copy(tr_v.at[db2], out_hbm.at[t, :, bt], out_sem)

    for t in range(3):
        fire_gather(t, t)

    def body(p, carry):
        for k in range(4):
            unit(p * 4 + k, k, k % 2)
        return carry

    lax.fori_loop(0, NT // 4, body, 0)
    for db in range(2):
        wait_out(db)


def kernel(x, emb):
    table = jnp.pad(emb, ((0, 0), (0, DP - D)))
    out5 = _emb_lookup(x.T.astype(jnp.int32), table)
    return jnp.transpose(out5, (2, 4, 0, 1, 3)).reshape(NB, NT, D)


# speedup vs baseline: 1.1037x; 1.1037x over previous
import functools

import jax
import jax.numpy as jnp
from jax import lax
from jax.experimental import pallas as pl
from jax.experimental.pallas import tpu as pltpu
from jax.experimental.pallas import tpu_sc as plsc

NB = 4096
NT = 200
D = 64
DP = 128
NW = 32
BT = 128
NBT = NB // BT

_mesh = plsc.VectorSubcoreMesh(core_axis_name="c", subcore_axis_name="s")


@functools.partial(
    pl.kernel,
    mesh=_mesh,
    compiler_params=pltpu.CompilerParams(needs_layout_passes=False),
    out_type=jax.ShapeDtypeStruct((NT, D // 8, NBT, 8, BT), jnp.float32),
    scratch_types=[
        pltpu.VMEM((NT, BT), jnp.int32),
        pltpu.VMEM((4, BT, DP), jnp.float32),
        pltpu.VMEM((2, D // 8, 8, BT), jnp.float32),
        pltpu.SemaphoreType.DMA,
        pltpu.SemaphoreType.DMA,
    ],
)
def _emb_lookup(xt_hbm, table_hbm, out_hbm, idx_v, rows_v, tr_v,
                in_sem, out_sem):
    bt = lax.axis_index("s") * 2 + lax.axis_index("c")
    b0 = bt * BT
    pltpu.sync_copy(xt_hbm.at[:, pl.ds(b0, BT)], idx_v)

    lanes = lax.iota(jnp.int32, 16)

    def fire_gather(t, db):
        pltpu.async_copy(table_hbm.at[idx_v.at[t]], rows_v.at[db], in_sem)

    def wait_rows(db):
        pltpu.make_async_copy(
            table_hbm.at[pl.ds(0, BT)], rows_v.at[db], in_sem
        ).wait()

    dt_vecs = [(dg * 16 + lanes) // 8 for dg in range(D // 16)]
    din_vecs = [(dg * 16 + lanes) % 8 for dg in range(D // 16)]

    def transpose(db4, db2):
        @plsc.parallel_loop(0, BT, 1, unroll=16)
        def _(b):
            bs = jnp.full((16,), b, jnp.int32)
            for dg in range(D // 16):
                v = rows_v[db4, b, pl.ds(dg * 16, 16)]
                plsc.store_scatter(
                    tr_v.at[db2], [dt_vecs[dg], din_vecs[dg], bs], v
                )

    def wait_out(db):
        pltpu.make_async_copy(
            out_hbm.at[0, :, 0], tr_v.at[db], out_sem
        ).wait()

    def unit(t, db4, db2):
        wait_rows(db4)

        @pl.when(t + 3 < NT)
        def _():
            fire_gather(t + 3, (db4 + 3) % 4)

        @pl.when(t >= 2)
        def _():
            wait_out(db2)

        transpose(db4, db2)
        pltpu.async_copy(tr_v.at[db2], out_hbm.at[t, :, bt], out_sem)

    for t in range(3):
        fire_gather(t, t)

    def body(p, carry):
        for k in range(4):
            unit(p * 4 + k, k, k % 2)
        return carry

    lax.fori_loop(0, NT // 4, body, 0)
    for db in range(2):
        wait_out(db)


def kernel(x, emb):
    table = jnp.pad(emb, ((0, 0), (0, DP - D)))
    out5 = _emb_lookup(x.T.astype(jnp.int32), table)
    return jnp.transpose(out5, (2, 4, 0, 1, 3)).reshape(NB, NT, D)
